# baseline (speedup 1.0000x reference)
import functools

import jax
import jax.numpy as jnp
from jax import lax
from jax.experimental import pallas as pl
from jax.experimental.pallas import tpu as pltpu
from jax.experimental.pallas import tpu_sc as plsc

D_MODEL = 64
SCALE = 8.0
CHUNK = 128
LANES = 16


@functools.partial(jax.jit, static_argnums=(2, 3, 4))
def _run(x_flat, table, num_cores, num_subcores, b_per_w):
    n_chunks = b_per_w // CHUNK
    nw = num_cores * num_subcores
    B = nw * b_per_w

    mesh = plsc.VectorSubcoreMesh(core_axis_name="c", subcore_axis_name="s")

    @functools.partial(
        pl.kernel,
        mesh=mesh,
        out_type=jax.ShapeDtypeStruct((B, D_MODEL), jnp.float32),
        scratch_types=[
            pltpu.VMEM((n_chunks, CHUNK), jnp.int32),
            pltpu.VMEM((CHUNK, D_MODEL), jnp.float32),
            pltpu.SemaphoreType.DMA,
        ],
        compiler_params=pltpu.CompilerParams(use_tc_tiling_on_sc=False),
    )
    def k(x_hbm, table_hbm, out_hbm, idx_v, rows_v, sem):
        wid = lax.axis_index("s") * num_cores + lax.axis_index("c")
        base = wid * b_per_w
        pltpu.sync_copy(x_hbm.at[wid], idx_v)

        def chunk_body(ci, carry):
            pltpu.async_copy(table_hbm.at[idx_v.at[ci]], rows_v, sem).wait()

            def scale_row(r, c):
                for j in range(D_MODEL // LANES):
                    sl = pl.ds(j * LANES, LANES)
                    rows_v[r, sl] = rows_v[r, sl] * SCALE
                return c

            lax.fori_loop(0, CHUNK, scale_row, 0, unroll=2)
            pltpu.sync_copy(
                rows_v, out_hbm.at[pl.ds(base + ci * CHUNK, CHUNK)]
            )
            return carry

        lax.fori_loop(0, n_chunks, chunk_body, 0)

    return k(x_flat.reshape(nw, n_chunks, CHUNK), table)


def kernel(x, table):
    B0, B1 = x.shape
    B = B0 * B1
    info = plsc.get_sparse_core_info()
    nw = info.num_cores * info.num_subcores
    b_per_w = B // nw
    out = _run(x.reshape(B), table, info.num_cores, info.num_subcores, b_per_w)
    return out.reshape(B0, B1, D_MODEL)


# speedup vs baseline: 1.0320x; 1.0320x over previous
import functools

import jax
import jax.numpy as jnp
from jax import lax
from jax.experimental import pallas as pl
from jax.experimental.pallas import tpu as pltpu
from jax.experimental.pallas import tpu_sc as plsc

D_MODEL = 64
SCALE = 8.0
CHUNK = 128
LANES = 16
NB = 5


@functools.partial(jax.jit, static_argnums=(2, 3, 4))
def _run(x_flat, table, num_cores, num_subcores, b_per_w):
    n_chunks = b_per_w // CHUNK
    n_groups = n_chunks // NB
    nw = num_cores * num_subcores
    B = nw * b_per_w

    mesh = plsc.VectorSubcoreMesh(core_axis_name="c", subcore_axis_name="s")

    @functools.partial(
        pl.kernel,
        mesh=mesh,
        out_type=jax.ShapeDtypeStruct((B, D_MODEL), jnp.float32),
        scratch_types=[
            pltpu.VMEM((n_chunks, CHUNK), jnp.int32),
            pltpu.VMEM((NB, CHUNK, D_MODEL), jnp.float32),
            pltpu.VMEM((NB, CHUNK, D_MODEL), jnp.float32),
            pltpu.SemaphoreType.DMA((NB,)),
            pltpu.SemaphoreType.DMA((NB,)),
        ],
        compiler_params=pltpu.CompilerParams(use_tc_tiling_on_sc=False),
    )
    def k(x_hbm, table_hbm, out_hbm, idx_v, gbuf, sbuf, gsem, ssem):
        wid = lax.axis_index("s") * num_cores + lax.axis_index("c")
        base = wid * b_per_w
        pltpu.sync_copy(x_hbm.at[wid], idx_v)

        def g_start(ci, b):
            pltpu.make_async_copy(
                table_hbm.at[idx_v.at[ci]], gbuf.at[b], gsem.at[b]
            ).start()

        def g_wait(b):
            pltpu.make_async_copy(
                table_hbm.at[idx_v.at[0]], gbuf.at[b], gsem.at[b]
            ).wait()

        def s_start(ci, b):
            pltpu.make_async_copy(
                sbuf.at[b],
                out_hbm.at[pl.ds(base + ci * CHUNK, CHUNK)],
                ssem.at[b],
            ).start()

        def s_wait(b):
            pltpu.make_async_copy(
                sbuf.at[b], out_hbm.at[pl.ds(base, CHUNK)], ssem.at[b]
            ).wait()

        def scale_chunk(b):
            def srow(r, c):
                for j in range(D_MODEL // LANES):
                    sl = pl.ds(j * LANES, LANES)
                    sbuf[b, r, sl] = gbuf[b, r, sl] * SCALE
                return c

            lax.fori_loop(0, CHUNK, srow, 0, unroll=2)

        for b in range(NB):
            g_start(b, b)

        def group(cg, c):
            for b in range(NB):
                ci = cg * NB + b
                g_wait(b)

                @pl.when(cg > 0)
                def _():
                    s_wait(b)

                scale_chunk(b)

                @pl.when(cg < n_groups - 1)
                def _():
                    g_start(ci + NB, b)

                s_start(ci, b)
            return c

        lax.fori_loop(0, n_groups, group, 0)

        for b in range(NB):
            s_wait(b)

    return k(x_flat.reshape(nw, n_chunks, CHUNK), table)


def kernel(x, table):
    B0, B1 = x.shape
    B = B0 * B1
    info = plsc.get_sparse_core_info()
    nw = info.num_cores * info.num_subcores
    b_per_w = B // nw
    out = _run(x.reshape(B), table, info.num_cores, info.num_subcores, b_per_w)
    return out.reshape(B0, B1, D_MODEL)
